# baseline (speedup 1.0000x reference)
import jax
import jax.numpy as jnp
from jax import lax
from jax.experimental import pallas as pl
from jax.experimental.pallas import tpu as pltpu

_BLOCK_ROWS = 25000


_SUB_ROWS = 1000


def _prompt_block_kernel(x_ref, t_ref, o_ref):
    t = t_ref[...]

    def body(j, carry):
        xs = x_ref[pl.ds(j * _SUB_ROWS, _SUB_ROWS), :]
        logitsT = lax.dot_general(
            t, xs, (((1,), (1,)), ((), ())),
            preferred_element_type=jnp.float32)
        m = jnp.max(logitsT, axis=0, keepdims=True)
        e = jnp.exp(logitsT - m)
        attnT = e / jnp.sum(e, axis=0, keepdims=True)
        prompt = lax.dot_general(
            attnT, t, (((0,), (0,)), ((), ())),
            preferred_element_type=jnp.float32)
        o_ref[pl.ds(j * _SUB_ROWS, _SUB_ROWS), :] = xs + prompt
        return carry

    lax.fori_loop(0, _BLOCK_ROWS // _SUB_ROWS, body, 0)


def kernel(x, token_embeds):
    n, d = x.shape
    t_num = token_embeds.shape[0]
    bn = _BLOCK_ROWS
    grid = (pl.cdiv(n, bn),)
    return pl.pallas_call(
        _prompt_block_kernel,
        grid=grid,
        in_specs=[
            pl.BlockSpec((bn, d), lambda i: (i, 0)),
            pl.BlockSpec((t_num, d), lambda i: (0, 0)),
        ],
        out_specs=pl.BlockSpec((bn, d), lambda i: (i, 0)),
        out_shape=jax.ShapeDtypeStruct((n, d), x.dtype),
        compiler_params=pltpu.CompilerParams(
            dimension_semantics=("parallel",)),
    )(x, token_embeds)


# speedup vs baseline: 1.8445x; 1.8445x over previous
import jax
import jax.numpy as jnp
from jax import lax
from jax.experimental import pallas as pl
from jax.experimental.pallas import tpu as pltpu

_BLOCK_ROWS = 10000


def _prompt_block_kernel(x_ref, t_ref, o_ref):
    x_blk = x_ref[...]
    t = t_ref[...]
    logitsT = lax.dot_general(
        t, x_blk, (((1,), (1,)), ((), ())),
        preferred_element_type=jnp.float32)
    m = jnp.max(logitsT, axis=0, keepdims=True)
    e = jnp.exp(logitsT - m)
    attnT = e / jnp.sum(e, axis=0, keepdims=True)
    prompt = lax.dot_general(
        attnT, t, (((0,), (0,)), ((), ())),
        preferred_element_type=jnp.float32)
    o_ref[...] = x_blk + prompt


def kernel(x, token_embeds):
    n, d = x.shape
    t_num = token_embeds.shape[0]
    bn = _BLOCK_ROWS
    grid = (pl.cdiv(n, bn),)
    return pl.pallas_call(
        _prompt_block_kernel,
        grid=grid,
        in_specs=[
            pl.BlockSpec((bn, d), lambda i: (i, 0)),
            pl.BlockSpec((t_num, d), lambda i: (0, 0)),
        ],
        out_specs=pl.BlockSpec((bn, d), lambda i: (i, 0)),
        out_shape=jax.ShapeDtypeStruct((n, d), x.dtype),
        compiler_params=pltpu.CompilerParams(
            dimension_semantics=("parallel",)),
    )(x, token_embeds)
